# Initial kernel scaffold; baseline (speedup 1.0000x reference)
import jax
import jax.numpy as jnp
from jax.experimental import pallas as pl


def kernel(x, tables):
    raise NotImplementedError("write your pallas kernel here")


# speedup vs baseline: 1.3457x; 1.3457x over previous
import functools

import jax
import jax.numpy as jnp
from jax import lax
from jax.experimental import pallas as pl
from jax.experimental.pallas import tpu as pltpu
from jax.experimental.pallas import tpu_sc as plsc

BATCH = 16384
N_CAT = 26
N_CONT = 26
VOCAB = 100000
EMB_DIM = 16
OUT_D = N_CAT * EMB_DIM + N_CONT

NW = 32
CHUNKS = 16
NB = BATCH // CHUNKS
TASKS_PER_W = (N_CAT * CHUNKS) // NW
NB_CONT = BATCH // NW

_mesh = plsc.VectorSubcoreMesh(core_axis_name="c", subcore_axis_name="s")


@functools.partial(
    pl.kernel,
    mesh=_mesh,
    out_type=jax.ShapeDtypeStruct((BATCH, OUT_D), jnp.float32),
    scratch_types=[
        pltpu.VMEM((NB,), jnp.int32),
        pltpu.VMEM((NB, EMB_DIM), jnp.float32),
        pltpu.VMEM((NB_CONT, N_CONT), jnp.float32),
        pltpu.SemaphoreType.DMA,
    ],
    compiler_params=pltpu.CompilerParams(use_tc_tiling_on_sc=False),
)
def _emb_kernel(xt_hbm, table_hbm, cont_hbm, out_hbm, idx_v, rows_v, cont_v, sem):
    w = lax.axis_index("s") * 2 + lax.axis_index("c")

    for k in range(TASKS_PER_W):
        t = w + k * NW
        f = t // CHUNKS
        c = t % CHUNKS
        b0 = c * NB
        pltpu.sync_copy(xt_hbm.at[f, pl.ds(b0, NB)], idx_v)
        offv = jnp.zeros((16,), jnp.int32) + f * VOCAB

        def _add_off(i, _):
            idx_v[pl.ds(i * 16, 16)] = idx_v[pl.ds(i * 16, 16)] + offv
            return 0

        lax.fori_loop(0, NB // 16, _add_off, 0)
        pltpu.async_copy(table_hbm.at[idx_v], rows_v, sem).wait()
        pltpu.sync_copy(
            rows_v, out_hbm.at[pl.ds(b0, NB), pl.ds(f * EMB_DIM, EMB_DIM)]
        )

    r0 = w * NB_CONT
    pltpu.sync_copy(cont_hbm.at[pl.ds(r0, NB_CONT)], cont_v)
    pltpu.sync_copy(
        cont_v, out_hbm.at[pl.ds(r0, NB_CONT), pl.ds(N_CAT * EMB_DIM, N_CONT)]
    )


def kernel(x, tables):
    xt = jnp.transpose(x[:, :N_CAT])
    cont = x[:, N_CAT:].astype(jnp.float32)
    table_flat = tables.reshape(N_CAT * VOCAB, EMB_DIM)
    return _emb_kernel(xt, table_flat, cont)


# speedup vs baseline: 7.5431x; 5.6054x over previous
import functools

import jax
import jax.numpy as jnp
from jax import lax
from jax.experimental import pallas as pl
from jax.experimental.pallas import tpu as pltpu
from jax.experimental.pallas import tpu_sc as plsc

BATCH = 16384
N_CAT = 26
N_CONT = 26
VOCAB = 100000
EMB_DIM = 16
OUT_D = N_CAT * EMB_DIM + N_CONT

NW = 32
N_ROWS = N_CAT * EMB_DIM
ROWS_PER_W = N_ROWS // NW
BCH = 4096
N_BCH = BATCH // BCH

_mesh = plsc.VectorSubcoreMesh(core_axis_name="c", subcore_axis_name="s")


@functools.partial(
    pl.kernel,
    mesh=_mesh,
    out_type=jax.ShapeDtypeStruct((OUT_D, BATCH), jnp.float32),
    scratch_types=[
        pltpu.VMEM((VOCAB,), jnp.float32),
        pltpu.VMEM((BCH,), jnp.int32),
        pltpu.VMEM((BCH,), jnp.float32),
    ],
    compiler_params=pltpu.CompilerParams(needs_layout_passes=False),
)
def _emb_kernel(tab_hbm, idx_hbm, xtc_hbm, out_hbm, vrow, idx_v, out_v):
    w = lax.axis_index("s") * 2 + lax.axis_index("c")

    for k in range(ROWS_PER_W):
        r = w * ROWS_PER_W + k
        f = r // EMB_DIM
        e = r % EMB_DIM
        pltpu.sync_copy(tab_hbm.at[f, e], vrow)
        for c in range(N_BCH):
            pltpu.sync_copy(idx_hbm.at[pl.ds(f * BATCH + c * BCH, BCH)], idx_v)

            def _gather(i, _):
                b = i * 128
                for u in range(8):
                    g = plsc.load_gather(vrow, [idx_v[pl.ds(b + u * 16, 16)]])
                    out_v[pl.ds(b + u * 16, 16)] = g
                return 0

            lax.fori_loop(0, BCH // 128, _gather, 0)
            pltpu.sync_copy(out_v, out_hbm.at[r, pl.ds(c * BCH, BCH)])

    @pl.when(w < N_CONT)
    def _cont():
        for c in range(N_BCH):
            pltpu.sync_copy(xtc_hbm.at[w, pl.ds(c * BCH, BCH)], idx_v)

            def _convert(i, _):
                b = i * 128
                for u in range(8):
                    s = pl.ds(b + u * 16, 16)
                    out_v[s] = idx_v[s].astype(jnp.float32)
                return 0

            lax.fori_loop(0, BCH // 128, _convert, 0)
            pltpu.sync_copy(out_v, out_hbm.at[N_ROWS + w, pl.ds(c * BCH, BCH)])


def kernel(x, tables):
    tab_t = jnp.transpose(tables, (0, 2, 1))
    idx1d = jnp.transpose(x[:, :N_CAT]).reshape(-1)
    xtc = jnp.transpose(x[:, N_CAT:])
    out_t = _emb_kernel(tab_t, idx1d, xtc)
    return jnp.transpose(out_t)
